# baseline (speedup 1.0000x reference)
import functools

import jax
import jax.numpy as jnp
from jax.experimental import pallas as pl
from jax.experimental.pallas import tpu as pltpu

_LANES = 128
_READ_STEPS = 2


def _count_block(x, acc, block_l):
    pos = jnp.where(x > 0.0, 1.0, 0.0)
    lanes = [None] * 8
    for k in range(block_l // _LANES):
        sl = pos[:, k * _LANES:(k + 1) * _LANES]
        j = k % 8
        lanes[j] = sl if lanes[j] is None else lanes[j] + sl
    while len(lanes) > 1:
        lanes = [a if b is None else a + b
                 for a, b in zip(lanes[0::2], lanes[1::2])]
    return acc + lanes[0]


def _colvec(row_ref, feat):
    sq = jnp.broadcast_to(row_ref[...], (feat, feat))
    r = jax.lax.broadcasted_iota(jnp.int32, (feat, feat), 0)
    c = jax.lax.broadcasted_iota(jnp.int32, (feat, feat), 1)
    return jnp.sum(jnp.where(r == c, sq, 0.0), axis=1, keepdims=True)


def _fused_body(x_ref, w_ref, b_ref, out_ref, acc_ref, *,
                block_l, inv_numel, feat):
    t = pl.program_id(1)

    @pl.when(t == 0)
    def _init():
        acc_ref[...] = jnp.zeros_like(acc_ref)

    @pl.when(t < _READ_STEPS)
    def _count():
        acc_ref[...] = _count_block(x_ref[...], acc_ref[...], block_l)

    @pl.when(t == _READ_STEPS)
    def _emit():
        total = jnp.sum(acc_ref[...])
        freq = total * inv_numel
        w_col = _colvec(w_ref, feat)
        b_col = _colvec(b_ref, feat)
        probs = jax.nn.sigmoid(freq * w_col + b_col)
        out_ref[...] = jnp.broadcast_to(probs, out_ref.shape)


def kernel(emb2d_0, emb2d_1, emb2d_2, emb2d_3, embeddings_3d, w_t, b):
    del emb2d_0, emb2d_1, emb2d_3, embeddings_3d
    current = emb2d_2
    n_rows, feat = current.shape
    numel = n_rows * feat

    num_cores = 2
    assert n_rows % (_LANES * num_cores * _READ_STEPS) == 0

    xt = current.T
    block_l = n_rows // _READ_STEPS
    half_l = n_rows // num_cores

    out_t = pl.pallas_call(
        functools.partial(_fused_body, block_l=block_l,
                          inv_numel=1.0 / float(numel), feat=feat),
        out_shape=jax.ShapeDtypeStruct((feat, n_rows), jnp.float32),
        grid=(num_cores, _READ_STEPS + 1),
        in_specs=[
            pl.BlockSpec((feat, block_l),
                         lambda c, t: (0, jnp.minimum(t, _READ_STEPS - 1))),
            pl.BlockSpec((1, feat), lambda c, t: (0, 0)),
            pl.BlockSpec((1, feat), lambda c, t: (0, 0)),
        ],
        out_specs=pl.BlockSpec((feat, half_l), lambda c, t: (0, c)),
        scratch_shapes=[pltpu.VMEM((feat, _LANES), jnp.float32)],
        compiler_params=pltpu.CompilerParams(
            dimension_semantics=("parallel", "arbitrary")),
    )(xt, w_t, b)

    return out_t.T


# speedup vs baseline: 1.0658x; 1.0658x over previous
import functools

import jax
import jax.numpy as jnp
from jax.experimental import pallas as pl
from jax.experimental.pallas import tpu as pltpu

_LANES = 128


def _count_body(x_ref, acc_ref, *, block_l):
    t = pl.program_id(1)

    @pl.when(t == 0)
    def _init():
        acc_ref[...] = jnp.zeros_like(acc_ref)

    x = x_ref[...]
    pos = jnp.where(x > 0.0, 1.0, 0.0)
    lanes = [None] * 8
    for k in range(block_l // _LANES):
        sl = pos[:, k * _LANES:(k + 1) * _LANES]
        j = k % 8
        lanes[j] = sl if lanes[j] is None else lanes[j] + sl
    while len(lanes) > 1:
        lanes = [a if b is None else a + b
                 for a, b in zip(lanes[0::2], lanes[1::2])]
    acc_ref[...] += lanes[0][None]


def _colvec(row_ref, feat):
    sq = jnp.broadcast_to(row_ref[...], (feat, feat))
    r = jax.lax.broadcasted_iota(jnp.int32, (feat, feat), 0)
    c = jax.lax.broadcasted_iota(jnp.int32, (feat, feat), 1)
    return jnp.sum(jnp.where(r == c, sq, 0.0), axis=1, keepdims=True)


def _finalize_broadcast_body(cnt_ref, w_ref, b_ref, out_ref, *,
                             inv_numel, feat):
    total = jnp.sum(cnt_ref[...])
    freq = total * inv_numel
    w_col = _colvec(w_ref, feat)
    b_col = _colvec(b_ref, feat)
    probs = jax.nn.sigmoid(freq * w_col + b_col)
    out_ref[...] = jnp.broadcast_to(probs, out_ref.shape)


def kernel(emb2d_0, emb2d_1, emb2d_2, emb2d_3, embeddings_3d, w_t, b):
    del emb2d_0, emb2d_1, emb2d_3, embeddings_3d
    current = emb2d_2
    n_rows, feat = current.shape
    numel = n_rows * feat

    assert n_rows % _LANES == 0, "table rows must be a multiple of 128"

    xt = current.T
    lane_tiles = n_rows // _LANES

    num_chunks = 4 if lane_tiles % 4 == 0 else (2 if lane_tiles % 2 == 0 else 1)
    tiles_per_chunk = lane_tiles // num_chunks

    def _steps(tiles):
        for s in (2,):
            if tiles % s == 0:
                return s
        return 1

    steps = 1
    block_l = (tiles_per_chunk // steps) * _LANES

    partial = pl.pallas_call(
        functools.partial(_count_body, block_l=block_l),
        out_shape=jax.ShapeDtypeStruct((num_chunks, feat, _LANES), jnp.float32),
        grid=(num_chunks, steps),
        in_specs=[pl.BlockSpec(
            (feat, block_l), lambda c, t, _s=steps: (0, c * _s + t))],
        out_specs=pl.BlockSpec((1, feat, _LANES), lambda c, t: (c, 0, 0)),
        compiler_params=pltpu.CompilerParams(
            dimension_semantics=("parallel", "arbitrary")),
    )(xt)

    steps2 = _steps(lane_tiles)
    block_l2 = (lane_tiles // steps2) * _LANES
    body = functools.partial(
        _finalize_broadcast_body, inv_numel=1.0 / float(numel), feat=feat)
    out_t = pl.pallas_call(
        body,
        out_shape=jax.ShapeDtypeStruct((feat, n_rows), jnp.float32),
        grid=(steps2,),
        in_specs=[
            pl.BlockSpec((num_chunks, feat, _LANES), lambda i: (0, 0, 0)),
            pl.BlockSpec((1, feat), lambda i: (0, 0)),
            pl.BlockSpec((1, feat), lambda i: (0, 0)),
        ],
        out_specs=pl.BlockSpec((feat, block_l2), lambda i: (0, i)),
        compiler_params=pltpu.CompilerParams(dimension_semantics=("parallel",)),
    )(partial, w_t, b)

    return out_t.T


# speedup vs baseline: 1.2431x; 1.1664x over previous
import functools

import jax
import jax.numpy as jnp
from jax.experimental import pallas as pl
from jax.experimental.pallas import tpu as pltpu

_LANES = 128


def _count_body(x_ref, acc_ref, *, block_l):
    t = pl.program_id(1)

    @pl.when(t == 0)
    def _init():
        acc_ref[...] = jnp.zeros_like(acc_ref)

    x = x_ref[...]
    pos = jnp.where(x > 0.0, 1.0, 0.0)
    lanes = [None] * 8
    for k in range(block_l // _LANES):
        sl = pos[:, k * _LANES:(k + 1) * _LANES]
        j = k % 8
        lanes[j] = sl if lanes[j] is None else lanes[j] + sl
    while len(lanes) > 1:
        lanes = [a if b is None else a + b
                 for a, b in zip(lanes[0::2], lanes[1::2])]
    acc_ref[...] += lanes[0][None]


def _colvec(row_ref, feat):
    sq = jnp.broadcast_to(row_ref[...], (feat, feat))
    r = jax.lax.broadcasted_iota(jnp.int32, (feat, feat), 0)
    c = jax.lax.broadcasted_iota(jnp.int32, (feat, feat), 1)
    return jnp.sum(jnp.where(r == c, sq, 0.0), axis=1, keepdims=True)


def _finalize_broadcast_body(cnt_ref, w_ref, b_ref, out_ref, *,
                             inv_numel, feat):
    total = jnp.sum(cnt_ref[...])
    freq = total * inv_numel
    w_col = _colvec(w_ref, feat)
    b_col = _colvec(b_ref, feat)
    probs = jax.nn.sigmoid(freq * w_col + b_col)
    out_ref[...] = jnp.broadcast_to(probs, out_ref.shape)


def kernel(emb2d_0, emb2d_1, emb2d_2, emb2d_3, embeddings_3d, w_t, b):
    del emb2d_0, emb2d_1, emb2d_3, embeddings_3d
    current = emb2d_2
    n_rows, feat = current.shape
    numel = n_rows * feat

    assert n_rows % _LANES == 0, "table rows must be a multiple of 128"

    xt = current.T
    lane_tiles = n_rows // _LANES

    num_chunks = 2 if lane_tiles % 2 == 0 else 1
    tiles_per_chunk = lane_tiles // num_chunks

    def _steps(tiles):
        for s in (2,):
            if tiles % s == 0:
                return s
        return 1

    steps = 1
    block_l = (tiles_per_chunk // steps) * _LANES

    partial = pl.pallas_call(
        functools.partial(_count_body, block_l=block_l),
        out_shape=jax.ShapeDtypeStruct((num_chunks, feat, _LANES), jnp.float32),
        grid=(num_chunks, steps),
        in_specs=[pl.BlockSpec(
            (feat, block_l), lambda c, t, _s=steps: (0, c * _s + t))],
        out_specs=pl.BlockSpec((1, feat, _LANES), lambda c, t: (c, 0, 0)),
        compiler_params=pltpu.CompilerParams(
            dimension_semantics=("parallel", "arbitrary")),
    )(xt)

    steps2 = _steps(lane_tiles)
    block_l2 = (lane_tiles // steps2) * _LANES
    body = functools.partial(
        _finalize_broadcast_body, inv_numel=1.0 / float(numel), feat=feat)
    out_t = pl.pallas_call(
        body,
        out_shape=jax.ShapeDtypeStruct((feat, n_rows), jnp.float32),
        grid=(steps2,),
        in_specs=[
            pl.BlockSpec((num_chunks, feat, _LANES), lambda i: (0, 0, 0)),
            pl.BlockSpec((1, feat), lambda i: (0, 0)),
            pl.BlockSpec((1, feat), lambda i: (0, 0)),
        ],
        out_specs=pl.BlockSpec((feat, block_l2), lambda i: (0, i)),
        compiler_params=pltpu.CompilerParams(dimension_semantics=("parallel",)),
    )(partial, w_t, b)

    return out_t.T
